# baseline (speedup 1.0000x reference)
import functools

import jax
import jax.numpy as jnp
from jax import lax
from jax.experimental import pallas as pl
from jax.experimental.pallas import tpu as pltpu
from jax.experimental.pallas import tpu_sc as plsc

N_USER = 1_000_000
N_ITEM = 1_000_000
K = 16
BATCH = 16384

NC = 2
NS = 16
NW = NC * NS
CHUNK = BATCH // NW
IDX_ROWS = CHUNK // 128
WAVES = CHUNK // 16

_MESH = plsc.VectorSubcoreMesh(
    core_axis_name="c", subcore_axis_name="s", num_cores=NC, num_subcores=NS
)


@functools.partial(
    pl.kernel,
    out_type=jax.ShapeDtypeStruct((BATCH,), jnp.float32),
    mesh=_MESH,
    compiler_params=pltpu.CompilerParams(
        needs_layout_passes=False, use_tc_tiling_on_sc=True),
    scratch_types=[
        pltpu.VMEM((IDX_ROWS, 128), jnp.int32),
        pltpu.VMEM((IDX_ROWS, 128), jnp.int32),
        pltpu.VMEM((2, 2, 8, 256), jnp.float32),
        pltpu.VMEM((2, 2, 8, 256), jnp.float32),
        pltpu.VMEM((CHUNK,), jnp.float32),
        pltpu.SemaphoreType.DMA,
        pltpu.SemaphoreType.DMA,
    ],
)
def _mf_kernel(uid_hbm, iid_hbm, uw_hbm, iw_hbm, out_hbm,
               uidx, iidx, ubuf, ibuf, outv, usem, isem):
    wid = lax.axis_index("c") * NS + lax.axis_index("s")
    lane = lax.iota(jnp.int32, 16)

    pltpu.sync_copy(uid_hbm.at[pl.ds(wid * IDX_ROWS, IDX_ROWS)], uidx)
    pltpu.sync_copy(iid_hbm.at[pl.ds(wid * IDX_ROWS, IDX_ROWS)], iidx)

    def idx_vecs(v):
        j = v // 8
        c = (v % 8) * 16
        return uidx[j, pl.ds(c, 16)], iidx[j, pl.ds(c, 16)]

    def fire(v, buf):
        uvec, ivec = idx_vecs(v)
        ub = uvec & ~15
        ib = ivec & ~15
        ubs = [pl.multiple_of(ub[p], 16) for p in range(16)]
        ibs = [pl.multiple_of(ib[p], 16) for p in range(16)]
        for kh in range(2):
            for p in range(16):
                pltpu.async_copy(
                    uw_hbm.at[kh, :, pl.ds(ubs[p], 16)],
                    ubuf.at[buf, kh, :, pl.ds(p * 16, 16)], usem)
            for p in range(16):
                pltpu.async_copy(
                    iw_hbm.at[kh, :, pl.ds(ibs[p], 16)],
                    ibuf.at[buf, kh, :, pl.ds(p * 16, 16)], isem)

    def drain():
        pltpu.make_async_copy(
            uw_hbm.at[:, :, pl.ds(0, 256)], ubuf.at[0], usem).wait()
        pltpu.make_async_copy(
            iw_hbm.at[:, :, pl.ds(0, 256)], ibuf.at[0], isem).wait()

    def compute(v, buf):
        uvec, ivec = idx_vecs(v)
        upos = lane * 16 + (uvec & 15)
        ipos = lane * 16 + (ivec & 15)
        acc = jnp.zeros((16,), jnp.float32)
        for kh in range(2):
            for kl in range(8):
                bb = jnp.full((16,), buf, jnp.int32)
                hh = jnp.full((16,), kh, jnp.int32)
                ll = jnp.full((16,), kl, jnp.int32)
                uu = plsc.load_gather(ubuf, [bb, hh, ll, upos])
                vv = plsc.load_gather(ibuf, [bb, hh, ll, ipos])
                acc = acc + uu * vv
        outv[pl.ds(v * 16, 16)] = acc

    fire(0, 0)

    def super_body(sv, carry):
        v0 = sv * 2
        fire(v0 + 1, 1)
        drain()
        compute(v0, 0)
        fire(v0 + 2, 0)
        drain()
        compute(v0 + 1, 1)
        return carry

    lax.fori_loop(0, WAVES // 2 - 1, super_body, 0, unroll=False)
    fire(WAVES - 1, 1)
    drain()
    compute(WAVES - 2, 0)
    drain()
    compute(WAVES - 1, 1)

    pltpu.sync_copy(outv, out_hbm.at[pl.ds(wid * CHUNK, CHUNK)])


def kernel(train_x, user_weight, item_weight):
    tx = train_x.astype(jnp.int32)
    uid = tx[:, 0].reshape(NW * IDX_ROWS, 128)
    iid = tx[:, 1].reshape(NW * IDX_ROWS, 128)
    uw3 = user_weight.T.reshape(2, 8, N_USER)
    iw3 = item_weight.T.reshape(2, 8, N_ITEM)
    return _mf_kernel(uid, iid, uw3, iw3)


# speedup vs baseline: 1.6088x; 1.6088x over previous
import functools

import jax
import jax.numpy as jnp
from jax import lax
from jax.experimental import pallas as pl
from jax.experimental.pallas import tpu as pltpu
from jax.experimental.pallas import tpu_sc as plsc

N_USER = 1_000_000
N_ITEM = 1_000_000
K = 16
BATCH = 16384

NC = 2
NS = 16
NW = NC * NS
CHUNK = BATCH // NW
IDX_ROWS = CHUNK // 128
WAVES = CHUNK // 16

_MESH = plsc.VectorSubcoreMesh(
    core_axis_name="c", subcore_axis_name="s", num_cores=NC, num_subcores=NS
)


@functools.partial(
    pl.kernel,
    out_type=jax.ShapeDtypeStruct((BATCH,), jnp.float32),
    mesh=_MESH,
    compiler_params=pltpu.CompilerParams(
        needs_layout_passes=False, use_tc_tiling_on_sc=True),
    scratch_types=[
        pltpu.VMEM((8, 128), jnp.int32),
        pltpu.VMEM((4, 2, 8, 256), jnp.float32),
        pltpu.VMEM((4, 2, 8, 256), jnp.float32),
        pltpu.VMEM((CHUNK,), jnp.float32),
        pltpu.SemaphoreType.DMA,
        pltpu.SemaphoreType.DMA,
    ],
)
def _mf_kernel(xid_hbm, uw_hbm, iw_hbm, out_hbm,
               xidx, ubuf, ibuf, outv, usem, isem):
    wid = lax.axis_index("c") * NS + lax.axis_index("s")
    lane = lax.iota(jnp.int32, 16)

    pltpu.sync_copy(xid_hbm.at[pl.ds(wid * 8, 8)], xidx)

    def idx_vecs(v):
        j = v // 8
        c = (v % 8) * 16
        return xidx[2 * j, pl.ds(c, 16)], xidx[2 * j + 1, pl.ds(c, 16)]

    def fire(v):
        uvec, ivec = idx_vecs(v)
        ub = uvec & ~15
        ib = ivec & ~15
        buf = lax.rem(v, 4)
        ubs = [pl.multiple_of(ub[p], 16) for p in range(16)]
        ibs = [pl.multiple_of(ib[p], 16) for p in range(16)]
        for kh in range(2):
            for p in range(16):
                pltpu.async_copy(
                    uw_hbm.at[kh, :, pl.ds(ubs[p], 16)],
                    ubuf.at[buf, kh, :, pl.ds(p * 16, 16)], usem)
            for p in range(16):
                pltpu.async_copy(
                    iw_hbm.at[kh, :, pl.ds(ibs[p], 16)],
                    ibuf.at[buf, kh, :, pl.ds(p * 16, 16)], isem)

    def drain():
        pltpu.make_async_copy(
            uw_hbm.at[:, :, pl.ds(0, 256)], ubuf.at[0], usem).wait()
        pltpu.make_async_copy(
            iw_hbm.at[:, :, pl.ds(0, 256)], ibuf.at[0], isem).wait()

    def compute(v):
        uvec, ivec = idx_vecs(v)
        upos = lane * 16 + (uvec & 15)
        ipos = lane * 16 + (ivec & 15)
        buf = lax.rem(v, 4)
        acc = jnp.zeros((16,), jnp.float32)
        for kh in range(2):
            for kl in range(8):
                bb = jnp.full((16,), buf, jnp.int32)
                hh = jnp.full((16,), kh, jnp.int32)
                ll = jnp.full((16,), kl, jnp.int32)
                uu = plsc.load_gather(ubuf, [bb, hh, ll, upos])
                vv = plsc.load_gather(ibuf, [bb, hh, ll, ipos])
                acc = acc + uu * vv
        outv[pl.ds(v * 16, 16)] = acc

    fire(0)
    fire(1)
    fire(2)

    def wave_body(v, carry):
        fire(v)
        drain()
        compute(v - 3)
        return carry

    lax.fori_loop(3, WAVES, wave_body, 0, unroll=False)
    for t in range(WAVES - 3, WAVES):
        drain()
        compute(t)

    pltpu.sync_copy(outv, out_hbm.at[pl.ds(wid * CHUNK, CHUNK)])


def kernel(train_x, user_weight, item_weight):
    tx = train_x.astype(jnp.int32)
    xid = tx.reshape(128, 128, 2).transpose(0, 2, 1).reshape(256, 128)
    uw3 = user_weight.T.reshape(2, 8, N_USER)
    iw3 = item_weight.T.reshape(2, 8, N_ITEM)
    return _mf_kernel(xid, uw3, iw3)
